# Initial kernel scaffold; baseline (speedup 1.0000x reference)
import jax
import jax.numpy as jnp
from jax.experimental import pallas as pl


def kernel(x, pe_weight):
    raise NotImplementedError("write your pallas kernel here")


# speedup vs baseline: 2.8518x; 2.8518x over previous
import jax
import jax.numpy as jnp
from jax.experimental import pallas as pl


_BT = 512


def _add_kernel(x_ref, pe_ref, o_ref):
    o_ref[...] = x_ref[...] + pe_ref[...]


def kernel(x, pe_weight):
    B, T, D = x.shape
    grid = (T // _BT, B)
    return pl.pallas_call(
        _add_kernel,
        grid=grid,
        in_specs=[
            pl.BlockSpec((1, _BT, D), lambda t, b: (b, t, 0)),
            pl.BlockSpec((_BT, D), lambda t, b: (t, 0)),
        ],
        out_specs=pl.BlockSpec((1, _BT, D), lambda t, b: (b, t, 0)),
        out_shape=jax.ShapeDtypeStruct((B, T, D), x.dtype),
    )(x, pe_weight[:T])


# speedup vs baseline: 3.1795x; 1.1149x over previous
import jax
import jax.numpy as jnp
from jax.experimental import pallas as pl


_BT = 1024


def _add_kernel(x_ref, pe_ref, o_ref):
    o_ref[...] = x_ref[...] + pe_ref[...]


def kernel(x, pe_weight):
    B, T, D = x.shape
    grid = (T // _BT, B)
    return pl.pallas_call(
        _add_kernel,
        grid=grid,
        in_specs=[
            pl.BlockSpec((1, _BT, D), lambda t, b: (b, t, 0)),
            pl.BlockSpec((_BT, D), lambda t, b: (t, 0)),
        ],
        out_specs=pl.BlockSpec((1, _BT, D), lambda t, b: (b, t, 0)),
        out_shape=jax.ShapeDtypeStruct((B, T, D), x.dtype),
    )(x, pe_weight[:T])


# speedup vs baseline: 3.3146x; 1.0425x over previous
import jax
import jax.numpy as jnp
from jax.experimental import pallas as pl


_BT = 2048


def _add_kernel(x_ref, pe_ref, o_ref):
    o_ref[...] = x_ref[...] + pe_ref[...]


def kernel(x, pe_weight):
    B, T, D = x.shape
    grid = (T // _BT, B)
    return pl.pallas_call(
        _add_kernel,
        grid=grid,
        in_specs=[
            pl.BlockSpec((1, _BT, D), lambda t, b: (b, t, 0)),
            pl.BlockSpec((_BT, D), lambda t, b: (t, 0)),
        ],
        out_specs=pl.BlockSpec((1, _BT, D), lambda t, b: (b, t, 0)),
        out_shape=jax.ShapeDtypeStruct((B, T, D), x.dtype),
    )(x, pe_weight[:T])
